# Initial kernel scaffold; baseline (speedup 1.0000x reference)
import jax
import jax.numpy as jnp
from jax.experimental import pallas as pl


def kernel(x, y, w_embed):
    raise NotImplementedError("write your pallas kernel here")


# speedup vs baseline: 12.8602x; 12.8602x over previous
import functools

import jax
import jax.numpy as jnp
from jax import lax
from jax.experimental import pallas as pl
from jax.experimental.pallas import tpu as pltpu
from jax.experimental.pallas import tpu_sc as plsc

_V = 1000


def _lse_body(w_ref, o_ref):
    w = w_ref[...]
    m = jnp.max(w, axis=1)
    s = jnp.sum(jnp.exp(w - m[:, None]), axis=1)
    o_ref[...] = m + jnp.log(s)


def _row_lse(w):
    return pl.pallas_call(
        _lse_body,
        out_shape=jax.ShapeDtypeStruct((w.shape[0],), jnp.float32),
    )(w)


def _make_sc_gather(tok, nc, ns):
    nw = nc * ns
    per_w = tok // nw
    assert tok % nw == 0 and per_w % 16 == 0
    ch = 80
    nchunk = per_w // ch
    nvec = per_w // 16
    mesh = plsc.VectorSubcoreMesh(core_axis_name="c", subcore_axis_name="s")

    @functools.partial(
        pl.kernel,
        out_type=jax.ShapeDtypeStruct((tok,), jnp.float32),
        mesh=mesh,
        scratch_types=[
            pltpu.VMEM((per_w,), jnp.int32),
            pltpu.VMEM((per_w,), jnp.int32),
            pltpu.VMEM((per_w,), jnp.int32),
            pltpu.VMEM((per_w,), jnp.float32),
            pltpu.VMEM((per_w,), jnp.float32),
            pltpu.SemaphoreType.DMA,
        ],
    )
    def sc_kernel(x_hbm, y_hbm, lse_hbm, w_hbm, out_hbm,
                  xv, yv, fidx, wxy, lseg, sem):
        wid = lax.axis_index("s") * nc + lax.axis_index("c")
        base = wid * per_w
        pltpu.sync_copy(x_hbm.at[pl.ds(base, per_w)], xv)
        pltpu.sync_copy(y_hbm.at[pl.ds(base, per_w)], yv)

        def fidx_body(v, carry):
            sl = pl.ds(v * 16, 16)
            fidx[sl] = xv[sl] * _V + yv[sl]
            return carry

        lax.fori_loop(0, nvec, fidx_body, 0)

        copies = []
        for c in range(nchunk):
            sl = pl.ds(c * ch, ch)
            copies.append(pltpu.async_copy(w_hbm.at[fidx.at[sl]], wxy.at[sl], sem))
            copies.append(pltpu.async_copy(lse_hbm.at[xv.at[sl]], lseg.at[sl], sem))
        for cp in copies:
            cp.wait()

        def sub_body(v, carry):
            sl = pl.ds(v * 16, 16)
            lseg[sl] = lseg[sl] - wxy[sl]
            return carry

        lax.fori_loop(0, nvec, sub_body, 0)

        pltpu.sync_copy(lseg, out_hbm.at[pl.ds(base, per_w)])

    return sc_kernel


def kernel(x, y, w_embed):
    b, t = x.shape
    tok = b * t
    info = plsc.get_sparse_core_info()
    lse = _row_lse(w_embed)
    sc = _make_sc_gather(tok, info.num_cores, info.num_subcores)
    loss = sc(x.reshape(-1).astype(jnp.int32),
              y.reshape(-1).astype(jnp.int32),
              lse,
              w_embed.reshape(-1))
    return loss.reshape(b, t)


# speedup vs baseline: 21.2271x; 1.6506x over previous
import functools

import jax
import jax.numpy as jnp
from jax import lax
from jax.experimental import pallas as pl
from jax.experimental.pallas import tpu as pltpu
from jax.experimental.pallas import tpu_sc as plsc

_V = 1000


def _table_body(w_ref, x_ref, y_ref, d_ref, fidx_ref):
    w = w_ref[...]
    m = jnp.max(w, axis=1)
    s = jnp.sum(jnp.exp(w - m[:, None]), axis=1)
    lse = m + jnp.log(s)
    d_ref[...] = lse[:, None] - w
    fidx_ref[...] = x_ref[...] * _V + y_ref[...]


def _tc_stage(w, x, y):
    return pl.pallas_call(
        _table_body,
        out_shape=(
            jax.ShapeDtypeStruct(w.shape, jnp.float32),
            jax.ShapeDtypeStruct(x.shape, jnp.int32),
        ),
    )(w, x, y)


def _make_sc_gather(tok, nc, ns):
    nw = nc * ns
    per_w = tok // nw
    assert tok % nw == 0 and per_w % 16 == 0
    ch = 80
    nchunk = per_w // ch
    mesh = plsc.VectorSubcoreMesh(core_axis_name="c", subcore_axis_name="s")

    @functools.partial(
        pl.kernel,
        out_type=jax.ShapeDtypeStruct((tok,), jnp.float32),
        mesh=mesh,
        scratch_types=[
            pltpu.VMEM((per_w,), jnp.int32),
            pltpu.VMEM((per_w,), jnp.float32),
            pltpu.SemaphoreType.DMA,
        ],
    )
    def sc_kernel(fidx_hbm, d_hbm, out_hbm, fidx, outv, sem):
        wid = lax.axis_index("s") * nc + lax.axis_index("c")
        base = wid * per_w
        pltpu.sync_copy(fidx_hbm.at[pl.ds(base, per_w)], fidx)
        copies = []
        for c in range(nchunk):
            sl = pl.ds(c * ch, ch)
            copies.append(pltpu.async_copy(d_hbm.at[fidx.at[sl]], outv.at[sl], sem))
        for cp in copies:
            cp.wait()
        pltpu.sync_copy(outv, out_hbm.at[pl.ds(base, per_w)])

    return sc_kernel


def kernel(x, y, w_embed):
    b, t = x.shape
    tok = b * t
    info = plsc.get_sparse_core_info()
    d_tab, fidx = _tc_stage(w_embed,
                            x.astype(jnp.int32),
                            y.astype(jnp.int32))
    sc = _make_sc_gather(tok, info.num_cores, info.num_subcores)
    loss = sc(fidx.reshape(-1), d_tab.reshape(-1))
    return loss.reshape(b, t)
